# Initial kernel scaffold; baseline (speedup 1.0000x reference)
import jax
import jax.numpy as jnp
from jax.experimental import pallas as pl


def kernel(x, somap):
    raise NotImplementedError("write your pallas kernel here")


# speedup vs baseline: 1.4400x; 1.4400x over previous
import jax
import jax.numpy as jnp
from jax import lax
from jax.experimental import pallas as pl
from jax.experimental.pallas import tpu as pltpu

_XS = 128
_M_BLK = 512


def _som_body(xt_ref, somap_ref, dists_ref, coords_ref, minval_ref, minidx_ref):
    m = pl.program_id(0)
    num_blocks = pl.num_programs(0)

    xt = xt_ref[...]
    somap_blk = somap_ref[...]

    mm = lax.dot_general(
        somap_blk, xt, (((1,), (0,)), ((), ())),
        preferred_element_type=jnp.float32)
    a2 = jnp.sum(somap_blk * somap_blk, axis=1, keepdims=True)
    b2 = jnp.sum(xt * xt, axis=0, keepdims=True)
    sq = a2 + b2 - 2.0 * mm
    d = jnp.sqrt(jnp.maximum(sq, 0.0))
    dists_ref[...] = d

    local_min = jnp.min(d, axis=0, keepdims=True)
    iota = lax.broadcasted_iota(jnp.int32, d.shape, 0)
    local_idx = jnp.min(
        jnp.where(d == local_min, iota, d.shape[0]),
        axis=0, keepdims=True) + m * _M_BLK

    @pl.when(m == 0)
    def _init():
        minval_ref[...] = local_min
        minidx_ref[...] = local_idx

    @pl.when(m > 0)
    def _merge():
        better = local_min < minval_ref[...]
        minval_ref[...] = jnp.where(better, local_min, minval_ref[...])
        minidx_ref[...] = jnp.where(better, local_idx, minidx_ref[...])

    @pl.when(m == num_blocks - 1)
    def _finish():
        bmu = minidx_ref[...]
        coords_ref[...] = jnp.concatenate(
            [bmu // _XS, bmu % _XS], axis=0).astype(jnp.int32)


def kernel(x, somap):
    n, dim = x.shape
    m_total = somap.shape[0]
    num_blocks = m_total // _M_BLK
    xt = x.T

    dists, coords_t = pl.pallas_call(
        _som_body,
        grid=(num_blocks,),
        in_specs=[
            pl.BlockSpec((dim, n), lambda i: (0, 0)),
            pl.BlockSpec((_M_BLK, dim), lambda i: (i, 0)),
        ],
        out_specs=[
            pl.BlockSpec((_M_BLK, n), lambda i: (i, 0)),
            pl.BlockSpec((2, n), lambda i: (0, 0)),
        ],
        out_shape=[
            jax.ShapeDtypeStruct((m_total, n), jnp.float32),
            jax.ShapeDtypeStruct((2, n), jnp.int32),
        ],
        scratch_shapes=[
            pltpu.VMEM((1, n), jnp.float32),
            pltpu.VMEM((1, n), jnp.int32),
        ],
    )(xt, somap)

    return (coords_t.T, dists)
